# baseline (speedup 1.0000x reference)
import functools

import jax
import jax.numpy as jnp
from jax.experimental import pallas as pl

_TOPK = 64


def _row_threshold(pre, k, nbits):
    int_min = jnp.int32(-2147483648)
    su = jax.lax.bitcast_convert_type(pre, jnp.int32)
    su = jnp.where(su < 0, su ^ jnp.int32(0x7FFFFFFF), su)

    def body(i, cur):
        cand = cur | jnp.left_shift(jnp.int32(1), 31 - i)
        thr = cand ^ int_min
        cnt = jnp.sum((su >= thr).astype(jnp.int32), axis=1, keepdims=True)
        return jnp.where(cnt >= k, cand, cur)

    cur = jax.lax.fori_loop(
        0, nbits, body, jnp.zeros((pre.shape[0], 1), jnp.int32)
    )
    lo = cur ^ int_min
    lo_b = jnp.where(lo < 0, lo ^ jnp.int32(0x7FFFFFFF), lo)
    return jax.lax.bitcast_convert_type(lo_b, jnp.float32)


def _enc_body(x_ref, we_ref, be_ref, bd_ref, pre_ref):
    xc = x_ref[...] - bd_ref[...]
    pre_ref[...] = (
        jnp.dot(xc, we_ref[...], preferred_element_type=jnp.float32)
        + be_ref[...]
    )


def _tau_body(pre_ref, tau_ref, *, k):
    tau_ref[...] = _row_threshold(pre_ref[...], k, 28)


def _maskdec_body(pre_ref, tau_ref, wd_ref, bd_ref, sp_ref, out_ref):
    l = pl.program_id(1)
    p = pre_ref[...]
    sp = jnp.where(p >= tau_ref[...], jnp.maximum(p, 0.0), 0.0)
    sp_ref[...] = sp

    @pl.when(l == 0)
    def _():
        out_ref[...] = jnp.broadcast_to(bd_ref[...], out_ref.shape)

    out_ref[...] += jnp.dot(sp, wd_ref[...], preferred_element_type=jnp.float32)


def kernel(x, W_enc, b_enc, W_dec, b_dec):
    T, D = x.shape
    L = W_enc.shape[1]

    te, le = min(1024, T), min(3072, L)
    pre = pl.pallas_call(
        _enc_body,
        grid=(L // le, T // te),
        in_specs=[
            pl.BlockSpec((te, D), lambda l, t: (t, 0)),
            pl.BlockSpec((D, le), lambda l, t: (0, l)),
            pl.BlockSpec((1, le), lambda l, t: (0, l)),
            pl.BlockSpec((1, D), lambda l, t: (0, 0)),
        ],
        out_specs=pl.BlockSpec((te, le), lambda l, t: (t, l)),
        out_shape=jax.ShapeDtypeStruct((T, L), jnp.float32),
    )(x, W_enc, b_enc.reshape(1, L), b_dec.reshape(1, D))

    tt = min(256, T)
    tau = pl.pallas_call(
        functools.partial(_tau_body, k=_TOPK),
        grid=(T // tt,),
        in_specs=[pl.BlockSpec((tt, L), lambda t: (t, 0))],
        out_specs=pl.BlockSpec((tt, 1), lambda t: (t, 0)),
        out_shape=jax.ShapeDtypeStruct((T, 1), jnp.float32),
    )(pre)

    td, ld = min(2048, T), min(768, L)
    sparse, recon = pl.pallas_call(
        _maskdec_body,
        grid=(T // td, L // ld),
        in_specs=[
            pl.BlockSpec((td, ld), lambda t, l: (t, l)),
            pl.BlockSpec((td, 1), lambda t, l: (t, 0)),
            pl.BlockSpec((ld, D), lambda t, l: (l, 0)),
            pl.BlockSpec((1, D), lambda t, l: (0, 0)),
        ],
        out_specs=[
            pl.BlockSpec((td, ld), lambda t, l: (t, l)),
            pl.BlockSpec((td, D), lambda t, l: (t, 0)),
        ],
        out_shape=[
            jax.ShapeDtypeStruct((T, L), jnp.float32),
            jax.ShapeDtypeStruct((T, D), jnp.float32),
        ],
    )(pre, tau, W_dec, b_dec.reshape(1, D))

    return (recon, sparse, pre)
